# baseline (speedup 1.0000x reference)
import functools

import jax
import jax.numpy as jnp
from jax import lax
from jax.experimental import pallas as pl
from jax.experimental.pallas import tpu as pltpu
from jax.experimental.pallas import tpu_sc as plsc

VOCAB = 1000000
DIM = 64
BATCH = 4096
SEQ = 200

NUM_WORKERS = 32
ROWS = BATCH * SEQ
PER_W = ROWS // NUM_WORKERS
CHUNK = 400
NCHUNK = PER_W // CHUNK
REPS = CHUNK // SEQ
NBUF = 3


def _pos_encoding():
    pos = jnp.arange(SEQ, dtype=jnp.float32)
    denom = 10000.0 ** jnp.linspace(0.0, 1.0, DIM)
    arg = pos[:, None] / denom[None, :]
    col = jnp.arange(DIM)
    return jnp.where(col[None, :] % 2 == 0, jnp.sin(arg), jnp.cos(arg))


def _body(idx_hbm, table_hbm, pos_hbm, out_hbm,
          idx_v, rows0, rows1, rows2, pos_v,
          sg0, sg1, sg2, so0, so1, so2):
    wid = lax.axis_index("s") * 2 + lax.axis_index("c")
    base = wid * PER_W

    pltpu.sync_copy(pos_hbm, pos_v)
    pltpu.sync_copy(idx_hbm.at[pl.ds(base, PER_W)], idx_v)

    rows = (rows0, rows1, rows2)
    sgs = (sg0, sg1, sg2)
    sos = (so0, so1, so2)

    def fire_gather(c):
        b = c % NBUF
        return pltpu.async_copy(
            table_hbm.at[idx_v.at[pl.ds(c * CHUNK, CHUNK)]],
            rows[b],
            sgs[b],
        )

    def add_chunk(c):
        rv = rows[c % NBUF]

        def add_row(r, carry2):
            p0 = pos_v[r, pl.ds(0, 16)]
            p1 = pos_v[r, pl.ds(16, 16)]
            p2 = pos_v[r, pl.ds(32, 16)]
            p3 = pos_v[r, pl.ds(48, 16)]
            for rep in range(REPS):
                q = rep * SEQ + r
                rv[q, pl.ds(0, 16)] = rv[q, pl.ds(0, 16)] + p0
                rv[q, pl.ds(16, 16)] = rv[q, pl.ds(16, 16)] + p1
                rv[q, pl.ds(32, 16)] = rv[q, pl.ds(32, 16)] + p2
                rv[q, pl.ds(48, 16)] = rv[q, pl.ds(48, 16)] + p3
            return carry2

        lax.fori_loop(0, SEQ, add_row, 0)

    def writeback(c):
        b = c % NBUF
        return pltpu.async_copy(
            rows[b], out_hbm.at[pl.ds(base + c * CHUNK, CHUNK)], sos[b])

    gathers = {0: fire_gather(0), 1: fire_gather(1)}
    out_copies = {}
    for c in range(NCHUNK):
        if c + 2 < NCHUNK:
            if c - 1 >= 0:
                out_copies.pop(c - 1).wait()
            gathers[c + 2] = fire_gather(c + 2)
        gathers.pop(c).wait()
        out_copies[c] = writeback(c)
    for c in (NCHUNK - 3, NCHUNK - 2, NCHUNK - 1):
        out_copies.pop(c).wait()


@jax.jit
def _run(idx_flat, table, pos):
    mesh = plsc.VectorSubcoreMesh(core_axis_name="c", subcore_axis_name="s")
    f = functools.partial(
        pl.kernel,
        mesh=mesh,
        out_type=jax.ShapeDtypeStruct((ROWS, DIM), jnp.float32),
        scratch_types=[
            pltpu.VMEM((PER_W,), jnp.int32),
            pltpu.VMEM((CHUNK, DIM), jnp.float32),
            pltpu.VMEM((CHUNK, DIM), jnp.float32),
            pltpu.VMEM((CHUNK, DIM), jnp.float32),
            pltpu.VMEM((SEQ, DIM), jnp.float32),
            pltpu.SemaphoreType.DMA,
            pltpu.SemaphoreType.DMA,
            pltpu.SemaphoreType.DMA,
            pltpu.SemaphoreType.DMA,
            pltpu.SemaphoreType.DMA,
            pltpu.SemaphoreType.DMA,
        ],
        compiler_params=pltpu.CompilerParams(use_tc_tiling_on_sc=False),
    )(_body)
    gathered = f(idx_flat, table, pos).reshape(BATCH, SEQ, DIM)
    return gathered + pos[None, :, :]


def kernel(input, table):
    idx_flat = input.reshape(ROWS)
    pos = _pos_encoding()
    return _run(idx_flat, table, pos)


# speedup vs baseline: 1.0971x; 1.0971x over previous
import functools

import jax
import jax.numpy as jnp
from jax import lax
from jax.experimental import pallas as pl
from jax.experimental.pallas import tpu as pltpu
from jax.experimental.pallas import tpu_sc as plsc

VOCAB = 1000000
DIM = 64
BATCH = 4096
SEQ = 200

NUM_WORKERS = 32
ROWS = BATCH * SEQ
PER_W = ROWS // NUM_WORKERS
CHUNK = 400
NCHUNK = PER_W // CHUNK
REPS = CHUNK // SEQ
NBUF = 3


def _pos_encoding():
    pos = jnp.arange(SEQ, dtype=jnp.float32)
    denom = 10000.0 ** jnp.linspace(0.0, 1.0, DIM)
    arg = pos[:, None] / denom[None, :]
    col = jnp.arange(DIM)
    return jnp.where(col[None, :] % 2 == 0, jnp.sin(arg), jnp.cos(arg))


def _body(idx_hbm, table_hbm, pos_hbm, out_hbm,
          idx_v, rows0, rows1, rows2, pos_v,
          sg0, sg1, sg2, so0, so1, so2):
    wid = lax.axis_index("s") * 2 + lax.axis_index("c")
    base = wid * PER_W

    pltpu.sync_copy(pos_hbm, pos_v)
    pltpu.sync_copy(idx_hbm.at[pl.ds(base, PER_W)], idx_v)

    rows = (rows0, rows1, rows2)
    sgs = (sg0, sg1, sg2)
    sos = (so0, so1, so2)

    def fire_gather(c):
        b = c % NBUF
        return pltpu.async_copy(
            table_hbm.at[idx_v.at[pl.ds(c * CHUNK, CHUNK)]],
            rows[b],
            sgs[b],
        )

    def add_chunk(c):
        rv = rows[c % NBUF]

        def add_row(r, carry2):
            p0 = pos_v[r, pl.ds(0, 16)]
            p1 = pos_v[r, pl.ds(16, 16)]
            p2 = pos_v[r, pl.ds(32, 16)]
            p3 = pos_v[r, pl.ds(48, 16)]
            for rep in range(REPS):
                q = rep * SEQ + r
                rv[q, pl.ds(0, 16)] = rv[q, pl.ds(0, 16)] + p0
                rv[q, pl.ds(16, 16)] = rv[q, pl.ds(16, 16)] + p1
                rv[q, pl.ds(32, 16)] = rv[q, pl.ds(32, 16)] + p2
                rv[q, pl.ds(48, 16)] = rv[q, pl.ds(48, 16)] + p3
            return carry2

        lax.fori_loop(0, SEQ, add_row, 0)

    def writeback(c):
        b = c % NBUF
        return pltpu.async_copy(
            rows[b], out_hbm.at[pl.ds(base + c * CHUNK, CHUNK)], sos[b])

    gathers = {0: fire_gather(0), 1: fire_gather(1)}
    out_copies = {}
    for c in range(NCHUNK):
        if c + 2 < NCHUNK:
            if c - 1 >= 0:
                out_copies.pop(c - 1).wait()
            gathers[c + 2] = fire_gather(c + 2)
        gathers.pop(c).wait()
        add_chunk(c)
        out_copies[c] = writeback(c)
    for c in (NCHUNK - 3, NCHUNK - 2, NCHUNK - 1):
        out_copies.pop(c).wait()


@jax.jit
def _run(idx_flat, table, pos):
    mesh = plsc.VectorSubcoreMesh(core_axis_name="c", subcore_axis_name="s")
    f = functools.partial(
        pl.kernel,
        mesh=mesh,
        out_type=jax.ShapeDtypeStruct((ROWS, DIM), jnp.float32),
        scratch_types=[
            pltpu.VMEM((PER_W,), jnp.int32),
            pltpu.VMEM((CHUNK, DIM), jnp.float32),
            pltpu.VMEM((CHUNK, DIM), jnp.float32),
            pltpu.VMEM((CHUNK, DIM), jnp.float32),
            pltpu.VMEM((SEQ, DIM), jnp.float32),
            pltpu.SemaphoreType.DMA,
            pltpu.SemaphoreType.DMA,
            pltpu.SemaphoreType.DMA,
            pltpu.SemaphoreType.DMA,
            pltpu.SemaphoreType.DMA,
            pltpu.SemaphoreType.DMA,
        ],
        compiler_params=pltpu.CompilerParams(use_tc_tiling_on_sc=False),
    )(_body)
    return f(idx_flat, table, pos).reshape(BATCH, SEQ, DIM)


def kernel(input, table):
    idx_flat = input.reshape(ROWS)
    pos = _pos_encoding()
    return _run(idx_flat, table, pos)
